# baseline (speedup 1.0000x reference)
import jax
import jax.numpy as jnp
from jax import lax
from jax.experimental import pallas as pl
from jax.experimental.pallas import tpu as pltpu
from jax.experimental.pallas import tpu_sc as plsc

N_CATS = 26
N_PLAIN = 22
N_TABLES = 23
VOCAB = 100000
DIM = 32
B, T = 1024, 50
GROUP_COLS = (22, 23, 24, 25)

NC, NS = 2, 16
NW = NC * NS
POS = B * T
CHUNK = POS // NW
ROWS128 = POS * DIM // 128


def _sc_body(xb_hbm, tbl_hbm, *refs):
    outs = refs[:N_CATS]
    ib0, ib1, ra, rb, isem, gsem, wsem = refs[N_CATS:]
    ib = (ib0, ib1)
    rows = (ra, rb)
    wid = lax.axis_index("s") * NC + lax.axis_index("c")
    base = wid * CHUNK

    def start_idx(f):
        return pltpu.async_copy(
            xb_hbm.at[pl.ds(f * POS + base, CHUNK)], ib[f % 2], isem)

    def start_gather(f):
        return pltpu.async_copy(tbl_hbm.at[ib[f % 2]], rows[f % 2], gsem)

    def start_wb(f):
        return pltpu.async_copy(
            rows[f % 2], outs[f].at[pl.ds(base, CHUNK)], wsem)

    g, w, di = {}, {}, {}
    start_idx(0).wait()
    g[0] = start_gather(0)
    di[1] = start_idx(1)
    for f in range(N_CATS):
        g[f].wait()
        if f + 1 < N_CATS:
            di[f + 1].wait()
            if f >= 1:
                w[f - 1].wait()
            g[f + 1] = start_gather(f + 1)
        if f + 2 < N_CATS:
            di[f + 2] = start_idx(f + 2)
        w[f] = start_wb(f)
    w[N_CATS - 2].wait()
    w[N_CATS - 1].wait()


def _tc_body(*refs):
    ins = refs[:N_CATS]
    outs = refs[N_CATS:]

    def tr(x):
        return jnp.transpose(x.reshape(128, DIM))

    for f in range(N_PLAIN):
        outs[f][0] = tr(ins[f][...])
    acc = tr(ins[N_PLAIN][...])
    for j in range(1, 4):
        acc = acc + tr(ins[N_PLAIN + j][...])
    outs[N_PLAIN][0] = acc


@jax.jit
def kernel(x, tables):
    xb = jnp.transpose(x, (2, 1, 0)).reshape(N_CATS, POS)
    tid = jnp.array(list(range(N_PLAIN)) + [N_PLAIN] * 4, dtype=jnp.int32)
    xb = (xb + tid[:, None] * VOCAB).reshape(-1)
    tbl = tables.reshape(N_TABLES * VOCAB, DIM)

    mesh = plsc.VectorSubcoreMesh(core_axis_name="c", subcore_axis_name="s")
    inter = pl.kernel(
        _sc_body,
        out_type=tuple(
            jax.ShapeDtypeStruct((POS, DIM), jnp.float32)
            for _ in range(N_CATS)
        ),
        mesh=mesh,
        scratch_types=[
            pltpu.VMEM((CHUNK,), jnp.int32),
            pltpu.VMEM((CHUNK,), jnp.int32),
            pltpu.VMEM((CHUNK, DIM), jnp.float32),
            pltpu.VMEM((CHUNK, DIM), jnp.float32),
            pltpu.SemaphoreType.DMA,
            pltpu.SemaphoreType.DMA,
            pltpu.SemaphoreType.DMA,
        ],
        compiler_params=pltpu.CompilerParams(use_tc_tiling_on_sc=False),
    )(xb, tbl)

    inter128 = tuple(o.reshape(ROWS128, 128) for o in inter)
    outs = pl.pallas_call(
        _tc_body,
        grid=(T, B // 128),
        in_specs=[
            pl.BlockSpec((DIM, 128), lambda t, bb: (t * 8 + bb, 0))
        ] * N_CATS,
        out_specs=[
            pl.BlockSpec((1, DIM, 128), lambda t, bb: (t, 0, bb))
        ] * N_TABLES,
        out_shape=[
            jax.ShapeDtypeStruct((T, DIM, B), jnp.float32)
        ] * N_TABLES,
    )(*inter128)
    return tuple(o.transpose(2, 0, 1) for o in outs)


# speedup vs baseline: 1.0496x; 1.0496x over previous
import jax
import jax.numpy as jnp
from jax import lax
from jax.experimental import pallas as pl
from jax.experimental.pallas import tpu as pltpu
from jax.experimental.pallas import tpu_sc as plsc

N_CATS = 26
N_PLAIN = 22
N_TABLES = 23
VOCAB = 100000
DIM = 32
B, T = 1024, 50
GROUP_COLS = (22, 23, 24, 25)

NC, NS = 2, 16
NW = NC * NS
POS = B * T
CHUNK = POS // NW
ROWS128 = POS * DIM // 128


def _sc_body(xb_hbm, tbl_hbm, *refs):
    outs = refs[:N_CATS]
    ib0, ib1, ra, rb, isem, gsem, wsem = refs[N_CATS:]
    ib = (ib0, ib1)
    rows = (ra, rb)
    wid = lax.axis_index("s") * NC + lax.axis_index("c")
    base = wid * CHUNK

    def start_idx(f):
        return pltpu.async_copy(
            xb_hbm.at[pl.ds(f * POS + base, CHUNK)], ib[f % 2], isem)

    def start_gather(f):
        return pltpu.async_copy(tbl_hbm.at[ib[f % 2]], rows[f % 2], gsem)

    def start_wb(f):
        return pltpu.async_copy(
            rows[f % 2], outs[f].at[pl.ds(base, CHUNK)], wsem)

    g, w, di = {}, {}, {}
    start_idx(0).wait()
    g[0] = start_gather(0)
    di[1] = start_idx(1)
    for f in range(N_CATS):
        g[f].wait()
        if f + 1 < N_CATS:
            di[f + 1].wait()
            if f >= 1:
                w[f - 1].wait()
            g[f + 1] = start_gather(f + 1)
        if f + 2 < N_CATS:
            di[f + 2] = start_idx(f + 2)
        w[f] = start_wb(f)
    w[N_CATS - 2].wait()
    w[N_CATS - 1].wait()


def _tc_body(*refs):
    ins = refs[:N_CATS]
    outs = refs[N_CATS:]

    def tr(x):
        return jnp.transpose(x.reshape(128, DIM))

    for bb in range(B // 128):
        sl = pl.ds(bb * DIM, DIM)
        ob = pl.ds(bb * 128, 128)
        for f in range(N_PLAIN):
            outs[f][0, :, ob] = tr(ins[f][sl, :])
        acc = tr(ins[N_PLAIN][sl, :])
        for j in range(1, 4):
            acc = acc + tr(ins[N_PLAIN + j][sl, :])
        outs[N_PLAIN][0, :, ob] = acc


@jax.jit
def kernel(x, tables):
    xb = jnp.transpose(x, (2, 1, 0)).reshape(N_CATS, POS)
    tid = jnp.array(list(range(N_PLAIN)) + [N_PLAIN] * 4, dtype=jnp.int32)
    xb = (xb + tid[:, None] * VOCAB).reshape(-1)
    tbl = tables.reshape(N_TABLES * VOCAB, DIM)

    mesh = plsc.VectorSubcoreMesh(core_axis_name="c", subcore_axis_name="s")
    inter = pl.kernel(
        _sc_body,
        out_type=tuple(
            jax.ShapeDtypeStruct((POS, DIM), jnp.float32)
            for _ in range(N_CATS)
        ),
        mesh=mesh,
        scratch_types=[
            pltpu.VMEM((CHUNK,), jnp.int32),
            pltpu.VMEM((CHUNK,), jnp.int32),
            pltpu.VMEM((CHUNK, DIM), jnp.float32),
            pltpu.VMEM((CHUNK, DIM), jnp.float32),
            pltpu.SemaphoreType.DMA,
            pltpu.SemaphoreType.DMA,
            pltpu.SemaphoreType.DMA,
        ],
        compiler_params=pltpu.CompilerParams(use_tc_tiling_on_sc=False),
    )(xb, tbl)

    inter128 = tuple(o.reshape(ROWS128, 128) for o in inter)
    outs = pl.pallas_call(
        _tc_body,
        grid=(T,),
        in_specs=[
            pl.BlockSpec((B * DIM // 128, 128), lambda t: (t, 0))
        ] * N_CATS,
        out_specs=[
            pl.BlockSpec((1, DIM, B), lambda t: (t, 0, 0))
        ] * N_TABLES,
        out_shape=[
            jax.ShapeDtypeStruct((T, DIM, B), jnp.float32)
        ] * N_TABLES,
    )(*inter128)
    return tuple(o.transpose(2, 0, 1) for o in outs)


# speedup vs baseline: 1.0518x; 1.0021x over previous
import jax
import jax.numpy as jnp
from jax import lax
from jax.experimental import pallas as pl
from jax.experimental.pallas import tpu as pltpu
from jax.experimental.pallas import tpu_sc as plsc

N_CATS = 26
N_PLAIN = 22
N_TABLES = 23
VOCAB = 100000
DIM = 32
B, T = 1024, 50
GROUP_COLS = (22, 23, 24, 25)

NC, NS = 2, 16
NW = NC * NS
POS = B * T
CHUNK = POS // NW
ROWS128 = POS * DIM // 128


def _sc_body(xb_hbm, tbl_hbm, *refs):
    outs = refs[:N_CATS]
    ib0, ib1, ra, rb, isem, gsem, wsem = refs[N_CATS:]
    ib = (ib0, ib1)
    rows = (ra, rb)
    wid = lax.axis_index("s") * NC + lax.axis_index("c")
    base = wid * CHUNK

    def start_idx(f):
        return pltpu.async_copy(
            xb_hbm.at[pl.ds(f * POS + base, CHUNK)], ib[f % 2], isem)

    def start_gather(f):
        t = min(f, N_PLAIN)
        return pltpu.async_copy(tbl_hbm.at[t].at[ib[f % 2]], rows[f % 2], gsem)

    def start_wb(f):
        return pltpu.async_copy(
            rows[f % 2], outs[f].at[pl.ds(base, CHUNK)], wsem)

    g, w, di = {}, {}, {}
    start_idx(0).wait()
    g[0] = start_gather(0)
    di[1] = start_idx(1)
    for f in range(N_CATS):
        g[f].wait()
        if f + 1 < N_CATS:
            di[f + 1].wait()
            if f >= 1:
                w[f - 1].wait()
            g[f + 1] = start_gather(f + 1)
        if f + 2 < N_CATS:
            di[f + 2] = start_idx(f + 2)
        w[f] = start_wb(f)
    w[N_CATS - 2].wait()
    w[N_CATS - 1].wait()


def _tc_body(*refs):
    ins = refs[:N_CATS]
    outs = refs[N_CATS:]

    def tr(x):
        return jnp.transpose(x.reshape(128, DIM))

    for bb in range(B // 128):
        sl = pl.ds(bb * DIM, DIM)
        ob = pl.ds(bb * 128, 128)
        for f in range(N_PLAIN):
            outs[f][0, :, ob] = tr(ins[f][sl, :])
        acc = tr(ins[N_PLAIN][sl, :])
        for j in range(1, 4):
            acc = acc + tr(ins[N_PLAIN + j][sl, :])
        outs[N_PLAIN][0, :, ob] = acc


@jax.jit
def kernel(x, tables):
    xb = jnp.transpose(x, (2, 1, 0)).reshape(N_CATS * POS)

    mesh = plsc.VectorSubcoreMesh(core_axis_name="c", subcore_axis_name="s")
    inter = pl.kernel(
        _sc_body,
        out_type=tuple(
            jax.ShapeDtypeStruct((POS, DIM), jnp.float32)
            for _ in range(N_CATS)
        ),
        mesh=mesh,
        scratch_types=[
            pltpu.VMEM((CHUNK,), jnp.int32),
            pltpu.VMEM((CHUNK,), jnp.int32),
            pltpu.VMEM((CHUNK, DIM), jnp.float32),
            pltpu.VMEM((CHUNK, DIM), jnp.float32),
            pltpu.SemaphoreType.DMA,
            pltpu.SemaphoreType.DMA,
            pltpu.SemaphoreType.DMA,
        ],
        compiler_params=pltpu.CompilerParams(use_tc_tiling_on_sc=False),
    )(xb, tables)

    inter128 = tuple(o.reshape(ROWS128, 128) for o in inter)
    outs = pl.pallas_call(
        _tc_body,
        grid=(T,),
        in_specs=[
            pl.BlockSpec((B * DIM // 128, 128), lambda t: (t, 0))
        ] * N_CATS,
        out_specs=[
            pl.BlockSpec((1, DIM, B), lambda t: (t, 0, 0))
        ] * N_TABLES,
        out_shape=[
            jax.ShapeDtypeStruct((T, DIM, B), jnp.float32)
        ] * N_TABLES,
    )(*inter128)
    return tuple(o.transpose(2, 0, 1) for o in outs)


# speedup vs baseline: 1.0818x; 1.0285x over previous
import jax
import jax.numpy as jnp
from jax import lax
from jax.experimental import pallas as pl
from jax.experimental.pallas import tpu as pltpu
from jax.experimental.pallas import tpu_sc as plsc

N_CATS = 26
N_PLAIN = 22
N_TABLES = 23
VOCAB = 100000
DIM = 32
B, T = 1024, 50
GROUP_COLS = (22, 23, 24, 25)

NC, NS = 2, 16
NW = NC * NS
POS = B * T
CHUNK = POS // NW


def _body(xb_hbm, tbl_hbm, *refs):
    outs = refs[:N_TABLES]
    ib0, ib1, ra, rb, isem, gsem, wsem = refs[N_TABLES:]
    ib = (ib0, ib1)
    rows = (ra, rb)
    wid = lax.axis_index("s") * NC + lax.axis_index("c")
    base = wid * CHUNK

    def start_idx(f):
        return pltpu.async_copy(
            xb_hbm.at[pl.ds(f * POS + base, CHUNK)], ib[f % 2], isem)

    def gslot(f):
        return 1 if f > N_PLAIN else f % 2

    def start_gather(f):
        return pltpu.async_copy(tbl_hbm.at[ib[f % 2]], rows[gslot(f)], gsem)

    def start_wb(f):
        return pltpu.async_copy(
            rows[f % 2], outs[f].at[pl.ds(base, CHUNK)], wsem)

    g, w, di = {}, {}, {}
    start_idx(0).wait()
    g[0] = start_gather(0)
    di[1] = start_idx(1)
    for f in range(N_PLAIN + 1):
        g[f].wait()
        if f + 1 < N_CATS:
            di[f + 1].wait()
            if f >= 1 and f - 1 < N_PLAIN:
                w[f - 1].wait()
            g[f + 1] = start_gather(f + 1)
        if f + 2 < N_CATS:
            di[f + 2] = start_idx(f + 2)
        if f < N_PLAIN:
            w[f] = start_wb(f)

    def add_row(i, _):
        for k in range(DIM // 16):
            sl = pl.ds(k * 16, 16)
            ra[i, sl] = ra[i, sl] + rb[i, sl]
        return 0

    for f in range(N_PLAIN + 1, N_CATS):
        g[f].wait()
        if f + 2 < N_CATS:
            di[f + 2] = start_idx(f + 2)
        lax.fori_loop(0, CHUNK, add_row, 0)
        if f + 1 < N_CATS:
            di[f + 1].wait()
            g[f + 1] = start_gather(f + 1)
    pltpu.sync_copy(ra, outs[N_PLAIN].at[pl.ds(base, CHUNK)])


@jax.jit
def kernel(x, tables):
    xb = jnp.transpose(x, (2, 1, 0)).reshape(N_CATS, POS)
    tid = jnp.array(list(range(N_PLAIN)) + [N_PLAIN] * 4, dtype=jnp.int32)
    xb = (xb + tid[:, None] * VOCAB).reshape(-1)
    tbl = tables.reshape(N_TABLES * VOCAB, DIM)

    mesh = plsc.VectorSubcoreMesh(core_axis_name="c", subcore_axis_name="s")
    out_type = tuple(
        jax.ShapeDtypeStruct((POS, DIM), jnp.float32) for _ in range(N_TABLES)
    )
    outs = pl.kernel(
        _body,
        out_type=out_type,
        mesh=mesh,
        scratch_types=[
            pltpu.VMEM((CHUNK,), jnp.int32),
            pltpu.VMEM((CHUNK,), jnp.int32),
            pltpu.VMEM((CHUNK, DIM), jnp.float32),
            pltpu.VMEM((CHUNK, DIM), jnp.float32),
            pltpu.SemaphoreType.DMA,
            pltpu.SemaphoreType.DMA,
            pltpu.SemaphoreType.DMA,
        ],
        compiler_params=pltpu.CompilerParams(use_tc_tiling_on_sc=False),
    )(xb, tbl)
    return tuple(
        o.reshape(T, B, DIM).transpose(1, 0, 2) for o in outs
    )
